# Initial kernel scaffold; baseline (speedup 1.0000x reference)
import jax
import jax.numpy as jnp
from jax.experimental import pallas as pl


def kernel(x, edge_attr, lin_in_w, lin_in_b, edge_emb_w, edge_emb_b, ep1_w, ep1_b, lin1_w, lin1_b, bn1_g, bn1_b, ep2_w, ep2_b, lin2_w, lin2_b, bn2_g, bn2_b, ep3_w, ep3_b, lin3_w, lin3_b, bn3_g, bn3_b, out_w, out_b, edge_index, batch):
    raise NotImplementedError("write your pallas kernel here")


# speedup vs baseline: 6.6909x; 6.6909x over previous
import functools

import jax
import jax.numpy as jnp
from jax import lax
from jax.experimental import pallas as pl
from jax.experimental.pallas import tpu as pltpu
from jax.experimental.pallas import tpu_sc as plsc

NN = 10000
EE = 320000
DD = 128
ED = 16
HH = 256
OUTD = 32
GG = 128

NC = 2
NS = 16
_STRIPE = 624
_LAST_STRIPE = NN - (NS - 1) * _STRIPE
_IND_CHUNK = 80

_sc_mesh = dict(core_axis_name="c", subcore_axis_name="s")


def _stripe_init(acc_s, zeros_hbm, s):
    @pl.when(s < NS - 1)
    def _():
        st = pl.multiple_of(s * _STRIPE, 8)
        pltpu.sync_copy(zeros_hbm.at[pl.ds(0, _STRIPE)], acc_s.at[pl.ds(st, _STRIPE)])

    @pl.when(s == NS - 1)
    def _():
        pltpu.sync_copy(zeros_hbm, acc_s.at[pl.ds((NS - 1) * _STRIPE, _LAST_STRIPE)])


def _stripe_writeback(acc_s, dst_hbm, s, coff):
    @pl.when(s < NS - 1)
    def _():
        st = pl.multiple_of(s * _STRIPE, 8)
        dst = pl.multiple_of(coff + s * _STRIPE, 8)
        pltpu.sync_copy(acc_s.at[pl.ds(st, _STRIPE)],
                        dst_hbm.at[pl.ds(dst, _STRIPE)])

    @pl.when(s == NS - 1)
    def _():
        dst = pl.multiple_of(coff + (NS - 1) * _STRIPE, 8)
        pltpu.sync_copy(acc_s.at[pl.ds((NS - 1) * _STRIPE, _LAST_STRIPE)],
                        dst_hbm.at[pl.ds(dst, _LAST_STRIPE)])


@functools.partial(
    pl.kernel,
    out_type=jax.ShapeDtypeStruct((2 * NN, 32), jnp.float32),
    mesh=plsc.VectorSubcoreMesh(**_sc_mesh),
    compiler_params=pltpu.CompilerParams(use_tc_tiling_on_sc=False),
    scratch_types=[
        pltpu.VMEM_SHARED((NN, 32), jnp.float32),
        pltpu.VMEM((1, _IND_CHUNK), jnp.int32),
        pltpu.VMEM((_IND_CHUNK, 32), jnp.float32),
    ],
)
def _edge_scatter(ea32_hbm, col_hbm, zeros_hbm, eacc_hbm, acc_s, cidx_v, buf_v):
    c = lax.axis_index("c")
    s = lax.axis_index("s")
    _stripe_init(acc_s, zeros_hbm, s)
    plsc.subcore_barrier()
    per_tile = EE // (NC * NS)
    base0 = (c * NS + s) * per_tile

    def chunk(i, carry):
        b = pl.multiple_of(base0 + i * _IND_CHUNK, 8)
        pltpu.sync_copy(col_hbm.at[pl.ds(b, _IND_CHUNK)], cidx_v.at[0])
        pltpu.sync_copy(ea32_hbm.at[pl.ds(b, _IND_CHUNK)], buf_v)
        pltpu.sync_copy(buf_v, acc_s.at[cidx_v.at[0]], add=True)
        return carry

    lax.fori_loop(0, per_tile // _IND_CHUNK, chunk, 0)
    plsc.subcore_barrier()
    _stripe_writeback(acc_s, eacc_hbm, s, c * NN)


@functools.partial(
    pl.kernel,
    out_type=jax.ShapeDtypeStruct((2 * NN, DD), jnp.float32),
    mesh=plsc.VectorSubcoreMesh(**_sc_mesh),
    compiler_params=pltpu.CompilerParams(use_tc_tiling_on_sc=False),
    scratch_types=[
        pltpu.VMEM_SHARED((NN, DD), jnp.float32),
        pltpu.VMEM((1, _IND_CHUNK), jnp.int32),
        pltpu.VMEM((1, _IND_CHUNK), jnp.int32),
        pltpu.VMEM((_IND_CHUNK, DD), jnp.float32),
        pltpu.SemaphoreType.DMA,
    ],
)
def _spmm(gst_hbm, row_hbm, col_hbm, zeros_hbm, oacc_hbm,
          acc_s, ridx_v, cidx_v, rows_v, sem):
    c = lax.axis_index("c")
    s = lax.axis_index("s")
    _stripe_init(acc_s, zeros_hbm, s)
    plsc.subcore_barrier()
    per_tile = EE // NS
    base0 = s * per_tile
    coff = c * NN

    def chunk(i, carry):
        b = pl.multiple_of(base0 + i * _IND_CHUNK, 8)
        pltpu.sync_copy(row_hbm.at[pl.ds(b, _IND_CHUNK)], ridx_v.at[0])
        pltpu.sync_copy(col_hbm.at[pl.ds(b, _IND_CHUNK)], cidx_v.at[0])

        def addoff(j, carry2):
            st = pl.multiple_of(j * 16, 16)
            ridx_v[0, pl.ds(st, 16)] = ridx_v[0, pl.ds(st, 16)] + coff
            return carry2

        lax.fori_loop(0, _IND_CHUNK // 16, addoff, 0)
        pltpu.async_copy(gst_hbm.at[ridx_v.at[0]], rows_v, sem).wait()
        pltpu.sync_copy(rows_v, acc_s.at[cidx_v.at[0]], add=True)
        return carry

    lax.fori_loop(0, per_tile // _IND_CHUNK, chunk, 0)
    plsc.subcore_barrier()
    _stripe_writeback(acc_s, oacc_hbm, s, coff)


_EBLK = 8000
_ESTEPS = EE // _EBLK
_NBLK = 2000
_NSTEPS = NN // _NBLK


def _onehot(seg_ref):
    return (seg_ref[...] == lax.broadcasted_iota(jnp.int32, (1, GG), 1)
            ).astype(jnp.float32)


def _graph_ranges(cnr):
    ii = lax.broadcasted_iota(jnp.int32, (GG, GG), 0)
    jj = lax.broadcasted_iota(jnp.int32, (GG, GG), 1)
    lt = (ii < jj).astype(jnp.float32)
    starts = jnp.dot(cnr, lt, preferred_element_type=jnp.float32)
    return starts, starts + cnr


def _onehot_edges(row_ref, cnr):
    starts, ends = _graph_ranges(cnr)
    r = row_ref[...].astype(jnp.float32)
    return ((r >= starts) & (r < ends)).astype(jnp.float32)


def _c00(a, b):
    return lax.dot_general(a, b, (((0,), (0,)), ((), ())),
                           preferred_element_type=jnp.float32)


def _estats_body(ea_ref, row_ref, cnr_ref, s_ref, q_ref, c_ref):
    i = pl.program_id(0)

    @pl.when(i == 0)
    def _():
        s_ref[...] = jnp.zeros_like(s_ref)
        q_ref[...] = jnp.zeros_like(q_ref)
        c_ref[...] = jnp.zeros_like(c_ref)

    ea = ea_ref[...]
    oh = _onehot_edges(row_ref, cnr_ref[...])
    s_ref[...] += _c00(oh, ea)
    q_ref[...] += _c00(oh, ea * ea)
    c_ref[...] += _c00(oh, jnp.ones((_EBLK, 1), jnp.float32))


_edge_stats = pl.pallas_call(
    _estats_body,
    grid=(_ESTEPS,),
    in_specs=[
        pl.BlockSpec((_EBLK, ED), lambda i: (i, 0)),
        pl.BlockSpec((_EBLK, 1), lambda i: (i, 0)),
        pl.BlockSpec((1, GG), lambda i: (0, 0)),
    ],
    out_specs=[
        pl.BlockSpec((GG, ED), lambda i: (0, 0)),
        pl.BlockSpec((GG, ED), lambda i: (0, 0)),
        pl.BlockSpec((GG, 1), lambda i: (0, 0)),
    ],
    out_shape=[
        jax.ShapeDtypeStruct((GG, ED), jnp.float32),
        jax.ShapeDtypeStruct((GG, ED), jnp.float32),
        jax.ShapeDtypeStruct((GG, 1), jnp.float32),
    ],
)


def _eembed_body(ea_ref, row_ref, cnr_ref, s_ref, q_ref, c_ref, w_ref, b_ref,
                 out_ref):
    cnt = jnp.maximum(c_ref[...], 1.0)
    em = s_ref[...] / cnt
    ev = q_ref[...] / cnt - em * em
    es = jnp.sqrt(jnp.maximum(ev, 0.0) + 1e-8)
    oh = _onehot_edges(row_ref, cnr_ref[...])
    emg = jnp.dot(oh, em, preferred_element_type=jnp.float32)
    esg = jnp.dot(oh, es, preferred_element_type=jnp.float32)
    ean = (ea_ref[...] - emg) / (esg + 1e-8)
    eae = jnp.dot(ean, w_ref[...], preferred_element_type=jnp.float32) + b_ref[...]
    out_ref[...] = jnp.concatenate(
        [eae, jnp.ones((_EBLK, 1), jnp.float32),
         jnp.zeros((_EBLK, 32 - ED - 1), jnp.float32)], axis=1)


_edge_embed = pl.pallas_call(
    _eembed_body,
    grid=(_ESTEPS,),
    in_specs=[
        pl.BlockSpec((_EBLK, ED), lambda i: (i, 0)),
        pl.BlockSpec((_EBLK, 1), lambda i: (i, 0)),
        pl.BlockSpec((1, GG), lambda i: (0, 0)),
        pl.BlockSpec((GG, ED), lambda i: (0, 0)),
        pl.BlockSpec((GG, ED), lambda i: (0, 0)),
        pl.BlockSpec((GG, 1), lambda i: (0, 0)),
        pl.BlockSpec((ED, ED), lambda i: (0, 0)),
        pl.BlockSpec((1, ED), lambda i: (0, 0)),
    ],
    out_specs=pl.BlockSpec((_EBLK, 32), lambda i: (i, 0)),
    out_shape=jax.ShapeDtypeStruct((EE, 32), jnp.float32),
)


def _nstats_body(x_ref, b_ref, sx_ref, qx_ref, cn_ref, cnr_ref):
    i = pl.program_id(0)

    @pl.when(i == 0)
    def _():
        sx_ref[...] = jnp.zeros_like(sx_ref)
        qx_ref[...] = jnp.zeros_like(qx_ref)
        cn_ref[...] = jnp.zeros_like(cn_ref)
        cnr_ref[...] = jnp.zeros_like(cnr_ref)

    x = x_ref[...]
    oh = _onehot(b_ref)
    sx_ref[...] += _c00(oh, x)
    qx_ref[...] += _c00(oh, x * x)
    cn_ref[...] += _c00(oh, jnp.ones((_NBLK, 1), jnp.float32))
    cnr_ref[...] += jnp.sum(oh, axis=0, keepdims=True)


_node_stats = pl.pallas_call(
    _nstats_body,
    grid=(_NSTEPS,),
    in_specs=[
        pl.BlockSpec((_NBLK, DD), lambda i: (i, 0)),
        pl.BlockSpec((_NBLK, 1), lambda i: (i, 0)),
    ],
    out_specs=[
        pl.BlockSpec((GG, DD), lambda i: (0, 0)),
        pl.BlockSpec((GG, DD), lambda i: (0, 0)),
        pl.BlockSpec((GG, 1), lambda i: (0, 0)),
        pl.BlockSpec((1, GG), lambda i: (0, 0)),
    ],
    out_shape=[
        jax.ShapeDtypeStruct((GG, DD), jnp.float32),
        jax.ShapeDtypeStruct((GG, DD), jnp.float32),
        jax.ShapeDtypeStruct((GG, 1), jnp.float32),
        jax.ShapeDtypeStruct((1, GG), jnp.float32),
    ],
)


def _deg_dis_eam(e0, e1):
    deg = e0[:, ED:ED + 1] + e1[:, ED:ED + 1]
    mask = (deg > 0).astype(jnp.float32)
    dis = jnp.where(deg > 0, lax.rsqrt(jnp.maximum(deg, 1.0)), 0.0)
    eam = (e0[:, :ED] + e1[:, :ED]) / jnp.maximum(deg, 1.0)
    return mask, dis, eam


def _layer1_body(x_ref, b_ref, sx_ref, qx_ref, cn_ref, e0_ref, e1_ref,
                 liw_ref, lib_ref, epw_ref, epb_ref, lw_ref, lb_ref,
                 g0_ref, g1_ref):
    cnt = jnp.maximum(cn_ref[...], 1.0)
    xm = sx_ref[...] / cnt
    xv = qx_ref[...] / cnt - xm * xm
    xs = jnp.sqrt(jnp.maximum(xv, 0.0) + 1e-8)
    oh = _onehot(b_ref)
    xmg = jnp.dot(oh, xm, preferred_element_type=jnp.float32)
    xsg = jnp.dot(oh, xs, preferred_element_type=jnp.float32)
    xn = (x_ref[...] - xmg) / (xsg + 1e-8)
    h = jnp.maximum(
        jnp.dot(xn, liw_ref[...], preferred_element_type=jnp.float32)
        + lib_ref[...], 0.0)
    mask, dis, eam = _deg_dis_eam(e0_ref[...], e1_ref[...])
    t = h + jnp.dot(eam, epw_ref[...], preferred_element_type=jnp.float32) \
        + mask * epb_ref[...]
    h2 = jnp.dot(t, lw_ref[...], preferred_element_type=jnp.float32) + lb_ref[...]
    g = dis * h2
    g0_ref[...] = g[:, :DD]
    g1_ref[...] = g[:, DD:]


_layer1 = pl.pallas_call(
    _layer1_body,
    grid=(_NSTEPS,),
    in_specs=[
        pl.BlockSpec((_NBLK, DD), lambda i: (i, 0)),
        pl.BlockSpec((_NBLK, 1), lambda i: (i, 0)),
        pl.BlockSpec((GG, DD), lambda i: (0, 0)),
        pl.BlockSpec((GG, DD), lambda i: (0, 0)),
        pl.BlockSpec((GG, 1), lambda i: (0, 0)),
        pl.BlockSpec((_NBLK, 32), lambda i: (i, 0)),
        pl.BlockSpec((_NBLK, 32), lambda i: (i + _NSTEPS, 0)),
        pl.BlockSpec((DD, HH), lambda i: (0, 0)),
        pl.BlockSpec((1, HH), lambda i: (0, 0)),
        pl.BlockSpec((ED, HH), lambda i: (0, 0)),
        pl.BlockSpec((1, HH), lambda i: (0, 0)),
        pl.BlockSpec((HH, HH), lambda i: (0, 0)),
        pl.BlockSpec((1, HH), lambda i: (0, 0)),
    ],
    out_specs=[
        pl.BlockSpec((_NBLK, DD), lambda i: (i, 0)),
        pl.BlockSpec((_NBLK, DD), lambda i: (i, 0)),
    ],
    out_shape=[
        jax.ShapeDtypeStruct((NN, DD), jnp.float32),
        jax.ShapeDtypeStruct((NN, DD), jnp.float32),
    ],
)


def _bnstats_body(a0_ref, a1_ref, e0_ref, e1_ref, sm_ref, sq_ref):
    i = pl.program_id(0)

    @pl.when(i == 0)
    def _():
        sm_ref[...] = jnp.zeros_like(sm_ref)
        sq_ref[...] = jnp.zeros_like(sq_ref)

    _, dis, _ = _deg_dis_eam(e0_ref[...], e1_ref[...])
    conv = jnp.concatenate([a0_ref[...], a1_ref[...]], axis=1) * dis
    sm_ref[...] += jnp.sum(conv, axis=0, keepdims=True)
    sq_ref[...] += jnp.sum(conv * conv, axis=0, keepdims=True)


_bnstats = pl.pallas_call(
    _bnstats_body,
    grid=(_NSTEPS,),
    in_specs=[
        pl.BlockSpec((_NBLK, DD), lambda i: (i, 0)),
        pl.BlockSpec((_NBLK, DD), lambda i: (i + _NSTEPS, 0)),
        pl.BlockSpec((_NBLK, 32), lambda i: (i, 0)),
        pl.BlockSpec((_NBLK, 32), lambda i: (i + _NSTEPS, 0)),
    ],
    out_specs=[
        pl.BlockSpec((1, HH), lambda i: (0, 0)),
        pl.BlockSpec((1, HH), lambda i: (0, 0)),
    ],
    out_shape=[
        jax.ShapeDtypeStruct((1, HH), jnp.float32),
        jax.ShapeDtypeStruct((1, HH), jnp.float32),
    ],
)


def _bn_relu(a0_ref, a1_ref, dis, sm_ref, sq_ref, bg_ref, bb_ref):
    conv = jnp.concatenate([a0_ref[...], a1_ref[...]], axis=1) * dis
    m = sm_ref[...] / NN
    v = sq_ref[...] / NN - m * m
    hn = (conv - m) / jnp.sqrt(jnp.maximum(v, 0.0) + 1e-5) * bg_ref[...] \
        + bb_ref[...]
    return jnp.maximum(hn, 0.0)


def _layer23_body(a0_ref, a1_ref, e0_ref, e1_ref, sm_ref, sq_ref,
                  bg_ref, bb_ref, epw_ref, epb_ref, lw_ref, lb_ref,
                  g0_ref, g1_ref):
    mask, dis, eam = _deg_dis_eam(e0_ref[...], e1_ref[...])
    h = _bn_relu(a0_ref, a1_ref, dis, sm_ref, sq_ref, bg_ref, bb_ref)
    t = h + jnp.dot(eam, epw_ref[...], preferred_element_type=jnp.float32) \
        + mask * epb_ref[...]
    h2 = jnp.dot(t, lw_ref[...], preferred_element_type=jnp.float32) + lb_ref[...]
    g = dis * h2
    g0_ref[...] = g[:, :DD]
    g1_ref[...] = g[:, DD:]


_layer23 = pl.pallas_call(
    _layer23_body,
    grid=(_NSTEPS,),
    in_specs=[
        pl.BlockSpec((_NBLK, DD), lambda i: (i, 0)),
        pl.BlockSpec((_NBLK, DD), lambda i: (i + _NSTEPS, 0)),
        pl.BlockSpec((_NBLK, 32), lambda i: (i, 0)),
        pl.BlockSpec((_NBLK, 32), lambda i: (i + _NSTEPS, 0)),
        pl.BlockSpec((1, HH), lambda i: (0, 0)),
        pl.BlockSpec((1, HH), lambda i: (0, 0)),
        pl.BlockSpec((1, HH), lambda i: (0, 0)),
        pl.BlockSpec((1, HH), lambda i: (0, 0)),
        pl.BlockSpec((ED, HH), lambda i: (0, 0)),
        pl.BlockSpec((1, HH), lambda i: (0, 0)),
        pl.BlockSpec((HH, HH), lambda i: (0, 0)),
        pl.BlockSpec((1, HH), lambda i: (0, 0)),
    ],
    out_specs=[
        pl.BlockSpec((_NBLK, DD), lambda i: (i, 0)),
        pl.BlockSpec((_NBLK, DD), lambda i: (i, 0)),
    ],
    out_shape=[
        jax.ShapeDtypeStruct((NN, DD), jnp.float32),
        jax.ShapeDtypeStruct((NN, DD), jnp.float32),
    ],
)


def _pool_body(a0_ref, a1_ref, e0_ref, e1_ref, sm_ref, sq_ref,
               bg_ref, bb_ref, b_ref, cn_ref, ow_ref, ob_ref,
               p_ref, out_ref):
    i = pl.program_id(0)

    @pl.when(i == 0)
    def _():
        p_ref[...] = jnp.zeros_like(p_ref)

    _, dis, _ = _deg_dis_eam(e0_ref[...], e1_ref[...])
    h = _bn_relu(a0_ref, a1_ref, dis, sm_ref, sq_ref, bg_ref, bb_ref)
    oh = _onehot(b_ref)
    p_ref[...] += _c00(oh, h)

    @pl.when(i == _NSTEPS - 1)
    def _():
        cnt = jnp.maximum(cn_ref[...], 1.0)
        pooled = p_ref[...] / cnt
        out_ref[...] = jnp.dot(pooled, ow_ref[...],
                               preferred_element_type=jnp.float32) + ob_ref[...]


_pool = pl.pallas_call(
    _pool_body,
    grid=(_NSTEPS,),
    in_specs=[
        pl.BlockSpec((_NBLK, DD), lambda i: (i, 0)),
        pl.BlockSpec((_NBLK, DD), lambda i: (i + _NSTEPS, 0)),
        pl.BlockSpec((_NBLK, 32), lambda i: (i, 0)),
        pl.BlockSpec((_NBLK, 32), lambda i: (i + _NSTEPS, 0)),
        pl.BlockSpec((1, HH), lambda i: (0, 0)),
        pl.BlockSpec((1, HH), lambda i: (0, 0)),
        pl.BlockSpec((1, HH), lambda i: (0, 0)),
        pl.BlockSpec((1, HH), lambda i: (0, 0)),
        pl.BlockSpec((_NBLK, 1), lambda i: (i, 0)),
        pl.BlockSpec((GG, 1), lambda i: (0, 0)),
        pl.BlockSpec((HH, OUTD), lambda i: (0, 0)),
        pl.BlockSpec((1, OUTD), lambda i: (0, 0)),
    ],
    out_specs=[
        pl.BlockSpec((GG, HH), lambda i: (0, 0)),
        pl.BlockSpec((GG, OUTD), lambda i: (0, 0)),
    ],
    out_shape=[
        jax.ShapeDtypeStruct((GG, HH), jnp.float32),
        jax.ShapeDtypeStruct((GG, OUTD), jnp.float32),
    ],
)


def kernel(x, edge_attr, lin_in_w, lin_in_b, edge_emb_w, edge_emb_b,
           ep1_w, ep1_b, lin1_w, lin1_b, bn1_g, bn1_b,
           ep2_w, ep2_b, lin2_w, lin2_b, bn2_g, bn2_b,
           ep3_w, ep3_b, lin3_w, lin3_b, bn3_g, bn3_b,
           out_w, out_b, edge_index, batch):
    row = edge_index[0]
    col = edge_index[1]
    b2d = batch.reshape(NN, 1)
    row2d = row.reshape(EE, 1)

    Sx, Qx, Cn, Cnr = _node_stats(x, b2d)
    S, Q, C = _edge_stats(edge_attr, row2d, Cnr)
    ea32 = _edge_embed(edge_attr, row2d, Cnr, S, Q, C,
                       edge_emb_w, edge_emb_b.reshape(1, ED))
    zeros32 = jnp.zeros((_LAST_STRIPE, 32), jnp.float32)
    eacc = _edge_scatter(ea32, col, zeros32)
    g0, g1 = _layer1(x, b2d, Sx, Qx, Cn, eacc, eacc,
                     lin_in_w, lin_in_b.reshape(1, HH),
                     ep1_w, ep1_b.reshape(1, HH),
                     lin1_w, lin1_b.reshape(1, HH))

    zerosD = jnp.zeros((_LAST_STRIPE, DD), jnp.float32)
    gst = jnp.concatenate([g0, g1], axis=0)
    oacc1 = _spmm(gst, row, col, zerosD)

    sm1, sq1 = _bnstats(oacc1, oacc1, eacc, eacc)
    g0, g1 = _layer23(oacc1, oacc1, eacc, eacc, sm1, sq1,
                      bn1_g.reshape(1, HH), bn1_b.reshape(1, HH),
                      ep2_w, ep2_b.reshape(1, HH),
                      lin2_w, lin2_b.reshape(1, HH))
    gst = jnp.concatenate([g0, g1], axis=0)
    oacc2 = _spmm(gst, row, col, zerosD)

    sm2, sq2 = _bnstats(oacc2, oacc2, eacc, eacc)
    g0, g1 = _layer23(oacc2, oacc2, eacc, eacc, sm2, sq2,
                      bn2_g.reshape(1, HH), bn2_b.reshape(1, HH),
                      ep3_w, ep3_b.reshape(1, HH),
                      lin3_w, lin3_b.reshape(1, HH))
    gst = jnp.concatenate([g0, g1], axis=0)
    oacc3 = _spmm(gst, row, col, zerosD)

    sm3, sq3 = _bnstats(oacc3, oacc3, eacc, eacc)
    _, out = _pool(oacc3, oacc3, eacc, eacc, sm3, sq3,
                   bn3_g.reshape(1, HH), bn3_b.reshape(1, HH),
                   b2d, Cn, out_w, out_b.reshape(1, OUTD))
    return out


# speedup vs baseline: 9.2960x; 1.3893x over previous
import functools

import jax
import jax.numpy as jnp
from jax import lax
from jax.experimental import pallas as pl
from jax.experimental.pallas import tpu as pltpu
from jax.experimental.pallas import tpu_sc as plsc

NN = 10000
EE = 320000
DD = 128
ED = 16
HH = 256
OUTD = 32
GG = 128

NC = 2
NS = 16
_STRIPE = 624
_LAST_STRIPE = NN - (NS - 1) * _STRIPE
_IND_CHUNK = 80

_sc_mesh = dict(core_axis_name="c", subcore_axis_name="s")


def _stripe_init(acc_s, zeros_hbm, s):
    @pl.when(s < NS - 1)
    def _():
        st = pl.multiple_of(s * _STRIPE, 8)
        pltpu.sync_copy(zeros_hbm.at[pl.ds(0, _STRIPE)], acc_s.at[pl.ds(st, _STRIPE)])

    @pl.when(s == NS - 1)
    def _():
        pltpu.sync_copy(zeros_hbm, acc_s.at[pl.ds((NS - 1) * _STRIPE, _LAST_STRIPE)])


def _stripe_writeback(acc_s, dst_hbm, s, coff):
    @pl.when(s < NS - 1)
    def _():
        st = pl.multiple_of(s * _STRIPE, 8)
        dst = pl.multiple_of(coff + s * _STRIPE, 8)
        pltpu.sync_copy(acc_s.at[pl.ds(st, _STRIPE)],
                        dst_hbm.at[pl.ds(dst, _STRIPE)])

    @pl.when(s == NS - 1)
    def _():
        dst = pl.multiple_of(coff + (NS - 1) * _STRIPE, 8)
        pltpu.sync_copy(acc_s.at[pl.ds((NS - 1) * _STRIPE, _LAST_STRIPE)],
                        dst_hbm.at[pl.ds(dst, _LAST_STRIPE)])


@functools.partial(
    pl.kernel,
    out_type=jax.ShapeDtypeStruct((2 * NN, 32), jnp.float32),
    mesh=plsc.VectorSubcoreMesh(**_sc_mesh),
    compiler_params=pltpu.CompilerParams(use_tc_tiling_on_sc=False),
    scratch_types=[
        pltpu.VMEM_SHARED((NN, 32), jnp.float32),
        pltpu.VMEM((1, _IND_CHUNK), jnp.int32),
        pltpu.VMEM((_IND_CHUNK, 32), jnp.float32),
    ],
)
def _edge_scatter(ea32_hbm, col_hbm, zeros_hbm, eacc_hbm, acc_s, cidx_v, buf_v):
    c = lax.axis_index("c")
    s = lax.axis_index("s")
    _stripe_init(acc_s, zeros_hbm, s)
    plsc.subcore_barrier()
    per_tile = EE // (NC * NS)
    base0 = (c * NS + s) * per_tile

    def chunk(i, carry):
        b = pl.multiple_of(base0 + i * _IND_CHUNK, 8)
        pltpu.sync_copy(col_hbm.at[pl.ds(b, _IND_CHUNK)], cidx_v.at[0])
        pltpu.sync_copy(ea32_hbm.at[pl.ds(b, _IND_CHUNK)], buf_v)
        pltpu.sync_copy(buf_v, acc_s.at[cidx_v.at[0]], add=True)
        return carry

    lax.fori_loop(0, per_tile // _IND_CHUNK, chunk, 0)
    plsc.subcore_barrier()
    _stripe_writeback(acc_s, eacc_hbm, s, c * NN)


@functools.partial(
    pl.kernel,
    out_type=jax.ShapeDtypeStruct((2 * NN, DD), jnp.float32),
    mesh=plsc.VectorSubcoreMesh(**_sc_mesh),
    compiler_params=pltpu.CompilerParams(use_tc_tiling_on_sc=False),
    scratch_types=[
        pltpu.VMEM_SHARED((NN, DD), jnp.float32),
        pltpu.VMEM((1, _IND_CHUNK), jnp.int32),
        pltpu.VMEM((1, _IND_CHUNK), jnp.int32),
        pltpu.VMEM((_IND_CHUNK, DD), jnp.float32),
        pltpu.SemaphoreType.DMA,
        pltpu.VMEM((1, _IND_CHUNK), jnp.int32),
        pltpu.VMEM((1, _IND_CHUNK), jnp.int32),
        pltpu.VMEM((_IND_CHUNK, DD), jnp.float32),
        pltpu.SemaphoreType.DMA,
    ],
)
def _spmm(gst_hbm, row_hbm, col_hbm, zeros_hbm, oacc_hbm,
          acc_s, ridx_a, cidx_a, rows_a, sem_a, ridx_b, cidx_b, rows_b, sem_b):
    c = lax.axis_index("c")
    s = lax.axis_index("s")
    _stripe_init(acc_s, zeros_hbm, s)
    plsc.subcore_barrier()
    per_tile = EE // NS
    base0 = s * per_tile
    coff = c * NN
    n_chunks = per_tile // _IND_CHUNK

    def load_idx(ridx_v, cidx_v, chunk_idx):
        b = pl.multiple_of(base0 + chunk_idx * _IND_CHUNK, 8)
        pltpu.sync_copy(row_hbm.at[pl.ds(b, _IND_CHUNK)], ridx_v.at[0])
        pltpu.sync_copy(col_hbm.at[pl.ds(b, _IND_CHUNK)], cidx_v.at[0])

        def addoff(j, carry2):
            st = pl.multiple_of(j * 16, 16)
            ridx_v[0, pl.ds(st, 16)] = ridx_v[0, pl.ds(st, 16)] + coff
            return carry2

        lax.fori_loop(0, _IND_CHUNK // 16, addoff, 0)

    def start_gather(ridx_v, rows_v, sem):
        pltpu.async_copy(gst_hbm.at[ridx_v.at[0]], rows_v, sem)

    def wait_gather(ridx_v, rows_v, sem):
        pltpu.make_async_copy(gst_hbm.at[ridx_v.at[0]], rows_v, sem).wait()

    load_idx(ridx_a, cidx_a, 0)
    start_gather(ridx_a, rows_a, sem_a)

    def pair(i, carry):
        load_idx(ridx_b, cidx_b, 2 * i + 1)
        start_gather(ridx_b, rows_b, sem_b)
        wait_gather(ridx_a, rows_a, sem_a)
        pltpu.sync_copy(rows_a, acc_s.at[cidx_a.at[0]], add=True)

        @pl.when(i < n_chunks // 2 - 1)
        def _():
            load_idx(ridx_a, cidx_a, 2 * i + 2)
            start_gather(ridx_a, rows_a, sem_a)

        wait_gather(ridx_b, rows_b, sem_b)
        pltpu.sync_copy(rows_b, acc_s.at[cidx_b.at[0]], add=True)
        return carry

    lax.fori_loop(0, n_chunks // 2, pair, 0)
    plsc.subcore_barrier()
    _stripe_writeback(acc_s, oacc_hbm, s, coff)


_EBLK = 8000
_ESTEPS = EE // _EBLK
_NBLK = 2000
_NSTEPS = NN // _NBLK


def _onehot(seg_ref):
    return (seg_ref[...] == lax.broadcasted_iota(jnp.int32, (1, GG), 1)
            ).astype(jnp.float32)


def _graph_ranges(cnr):
    ii = lax.broadcasted_iota(jnp.int32, (GG, GG), 0)
    jj = lax.broadcasted_iota(jnp.int32, (GG, GG), 1)
    lt = (ii < jj).astype(jnp.float32)
    starts = jnp.dot(cnr, lt, preferred_element_type=jnp.float32)
    return starts, starts + cnr


def _onehot_edges(row_ref, cnr):
    starts, ends = _graph_ranges(cnr)
    r = row_ref[...].astype(jnp.float32)
    return ((r >= starts) & (r < ends)).astype(jnp.float32)


def _c00(a, b):
    return lax.dot_general(a, b, (((0,), (0,)), ((), ())),
                           preferred_element_type=jnp.float32)


def _estats_body(ea_ref, row_ref, cnr_ref, s_ref, q_ref, c_ref):
    i = pl.program_id(0)

    @pl.when(i == 0)
    def _():
        s_ref[...] = jnp.zeros_like(s_ref)
        q_ref[...] = jnp.zeros_like(q_ref)
        c_ref[...] = jnp.zeros_like(c_ref)

    ea = ea_ref[...]
    oh = _onehot_edges(row_ref, cnr_ref[...])
    s_ref[...] += _c00(oh, ea)
    q_ref[...] += _c00(oh, ea * ea)
    c_ref[...] += _c00(oh, jnp.ones((_EBLK, 1), jnp.float32))


_edge_stats = pl.pallas_call(
    _estats_body,
    grid=(_ESTEPS,),
    in_specs=[
        pl.BlockSpec((_EBLK, ED), lambda i: (i, 0)),
        pl.BlockSpec((_EBLK, 1), lambda i: (i, 0)),
        pl.BlockSpec((1, GG), lambda i: (0, 0)),
    ],
    out_specs=[
        pl.BlockSpec((GG, ED), lambda i: (0, 0)),
        pl.BlockSpec((GG, ED), lambda i: (0, 0)),
        pl.BlockSpec((GG, 1), lambda i: (0, 0)),
    ],
    out_shape=[
        jax.ShapeDtypeStruct((GG, ED), jnp.float32),
        jax.ShapeDtypeStruct((GG, ED), jnp.float32),
        jax.ShapeDtypeStruct((GG, 1), jnp.float32),
    ],
)


def _eembed_body(ea_ref, row_ref, cnr_ref, s_ref, q_ref, c_ref, w_ref, b_ref,
                 out_ref):
    cnt = jnp.maximum(c_ref[...], 1.0)
    em = s_ref[...] / cnt
    ev = q_ref[...] / cnt - em * em
    es = jnp.sqrt(jnp.maximum(ev, 0.0) + 1e-8)
    oh = _onehot_edges(row_ref, cnr_ref[...])
    emg = jnp.dot(oh, em, preferred_element_type=jnp.float32)
    esg = jnp.dot(oh, es, preferred_element_type=jnp.float32)
    ean = (ea_ref[...] - emg) / (esg + 1e-8)
    eae = jnp.dot(ean, w_ref[...], preferred_element_type=jnp.float32) + b_ref[...]
    out_ref[...] = jnp.concatenate(
        [eae, jnp.ones((_EBLK, 1), jnp.float32),
         jnp.zeros((_EBLK, 32 - ED - 1), jnp.float32)], axis=1)


_edge_embed = pl.pallas_call(
    _eembed_body,
    grid=(_ESTEPS,),
    in_specs=[
        pl.BlockSpec((_EBLK, ED), lambda i: (i, 0)),
        pl.BlockSpec((_EBLK, 1), lambda i: (i, 0)),
        pl.BlockSpec((1, GG), lambda i: (0, 0)),
        pl.BlockSpec((GG, ED), lambda i: (0, 0)),
        pl.BlockSpec((GG, ED), lambda i: (0, 0)),
        pl.BlockSpec((GG, 1), lambda i: (0, 0)),
        pl.BlockSpec((ED, ED), lambda i: (0, 0)),
        pl.BlockSpec((1, ED), lambda i: (0, 0)),
    ],
    out_specs=pl.BlockSpec((_EBLK, 32), lambda i: (i, 0)),
    out_shape=jax.ShapeDtypeStruct((EE, 32), jnp.float32),
)


def _nstats_body(x_ref, b_ref, sx_ref, qx_ref, cn_ref, cnr_ref):
    i = pl.program_id(0)

    @pl.when(i == 0)
    def _():
        sx_ref[...] = jnp.zeros_like(sx_ref)
        qx_ref[...] = jnp.zeros_like(qx_ref)
        cn_ref[...] = jnp.zeros_like(cn_ref)
        cnr_ref[...] = jnp.zeros_like(cnr_ref)

    x = x_ref[...]
    oh = _onehot(b_ref)
    sx_ref[...] += _c00(oh, x)
    qx_ref[...] += _c00(oh, x * x)
    cn_ref[...] += _c00(oh, jnp.ones((_NBLK, 1), jnp.float32))
    cnr_ref[...] += jnp.sum(oh, axis=0, keepdims=True)


_node_stats = pl.pallas_call(
    _nstats_body,
    grid=(_NSTEPS,),
    in_specs=[
        pl.BlockSpec((_NBLK, DD), lambda i: (i, 0)),
        pl.BlockSpec((_NBLK, 1), lambda i: (i, 0)),
    ],
    out_specs=[
        pl.BlockSpec((GG, DD), lambda i: (0, 0)),
        pl.BlockSpec((GG, DD), lambda i: (0, 0)),
        pl.BlockSpec((GG, 1), lambda i: (0, 0)),
        pl.BlockSpec((1, GG), lambda i: (0, 0)),
    ],
    out_shape=[
        jax.ShapeDtypeStruct((GG, DD), jnp.float32),
        jax.ShapeDtypeStruct((GG, DD), jnp.float32),
        jax.ShapeDtypeStruct((GG, 1), jnp.float32),
        jax.ShapeDtypeStruct((1, GG), jnp.float32),
    ],
)


def _deg_dis_eam(e0, e1):
    deg = e0[:, ED:ED + 1] + e1[:, ED:ED + 1]
    mask = (deg > 0).astype(jnp.float32)
    dis = jnp.where(deg > 0, lax.rsqrt(jnp.maximum(deg, 1.0)), 0.0)
    eam = (e0[:, :ED] + e1[:, :ED]) / jnp.maximum(deg, 1.0)
    return mask, dis, eam


def _layer1_body(x_ref, b_ref, sx_ref, qx_ref, cn_ref, e0_ref, e1_ref,
                 liw_ref, lib_ref, epw_ref, epb_ref, lw_ref, lb_ref,
                 g0_ref, g1_ref):
    cnt = jnp.maximum(cn_ref[...], 1.0)
    xm = sx_ref[...] / cnt
    xv = qx_ref[...] / cnt - xm * xm
    xs = jnp.sqrt(jnp.maximum(xv, 0.0) + 1e-8)
    oh = _onehot(b_ref)
    xmg = jnp.dot(oh, xm, preferred_element_type=jnp.float32)
    xsg = jnp.dot(oh, xs, preferred_element_type=jnp.float32)
    xn = (x_ref[...] - xmg) / (xsg + 1e-8)
    h = jnp.maximum(
        jnp.dot(xn, liw_ref[...], preferred_element_type=jnp.float32)
        + lib_ref[...], 0.0)
    mask, dis, eam = _deg_dis_eam(e0_ref[...], e1_ref[...])
    t = h + jnp.dot(eam, epw_ref[...], preferred_element_type=jnp.float32) \
        + mask * epb_ref[...]
    h2 = jnp.dot(t, lw_ref[...], preferred_element_type=jnp.float32) + lb_ref[...]
    g = dis * h2
    g0_ref[...] = g[:, :DD]
    g1_ref[...] = g[:, DD:]


_layer1 = pl.pallas_call(
    _layer1_body,
    grid=(_NSTEPS,),
    in_specs=[
        pl.BlockSpec((_NBLK, DD), lambda i: (i, 0)),
        pl.BlockSpec((_NBLK, 1), lambda i: (i, 0)),
        pl.BlockSpec((GG, DD), lambda i: (0, 0)),
        pl.BlockSpec((GG, DD), lambda i: (0, 0)),
        pl.BlockSpec((GG, 1), lambda i: (0, 0)),
        pl.BlockSpec((_NBLK, 32), lambda i: (i, 0)),
        pl.BlockSpec((_NBLK, 32), lambda i: (i + _NSTEPS, 0)),
        pl.BlockSpec((DD, HH), lambda i: (0, 0)),
        pl.BlockSpec((1, HH), lambda i: (0, 0)),
        pl.BlockSpec((ED, HH), lambda i: (0, 0)),
        pl.BlockSpec((1, HH), lambda i: (0, 0)),
        pl.BlockSpec((HH, HH), lambda i: (0, 0)),
        pl.BlockSpec((1, HH), lambda i: (0, 0)),
    ],
    out_specs=[
        pl.BlockSpec((_NBLK, DD), lambda i: (i, 0)),
        pl.BlockSpec((_NBLK, DD), lambda i: (i, 0)),
    ],
    out_shape=[
        jax.ShapeDtypeStruct((NN, DD), jnp.float32),
        jax.ShapeDtypeStruct((NN, DD), jnp.float32),
    ],
)


def _bnstats_body(a0_ref, a1_ref, e0_ref, e1_ref, sm_ref, sq_ref):
    i = pl.program_id(0)

    @pl.when(i == 0)
    def _():
        sm_ref[...] = jnp.zeros_like(sm_ref)
        sq_ref[...] = jnp.zeros_like(sq_ref)

    _, dis, _ = _deg_dis_eam(e0_ref[...], e1_ref[...])
    conv = jnp.concatenate([a0_ref[...], a1_ref[...]], axis=1) * dis
    sm_ref[...] += jnp.sum(conv, axis=0, keepdims=True)
    sq_ref[...] += jnp.sum(conv * conv, axis=0, keepdims=True)


_bnstats = pl.pallas_call(
    _bnstats_body,
    grid=(_NSTEPS,),
    in_specs=[
        pl.BlockSpec((_NBLK, DD), lambda i: (i, 0)),
        pl.BlockSpec((_NBLK, DD), lambda i: (i + _NSTEPS, 0)),
        pl.BlockSpec((_NBLK, 32), lambda i: (i, 0)),
        pl.BlockSpec((_NBLK, 32), lambda i: (i + _NSTEPS, 0)),
    ],
    out_specs=[
        pl.BlockSpec((1, HH), lambda i: (0, 0)),
        pl.BlockSpec((1, HH), lambda i: (0, 0)),
    ],
    out_shape=[
        jax.ShapeDtypeStruct((1, HH), jnp.float32),
        jax.ShapeDtypeStruct((1, HH), jnp.float32),
    ],
)


def _bn_relu(a0_ref, a1_ref, dis, sm_ref, sq_ref, bg_ref, bb_ref):
    conv = jnp.concatenate([a0_ref[...], a1_ref[...]], axis=1) * dis
    m = sm_ref[...] / NN
    v = sq_ref[...] / NN - m * m
    hn = (conv - m) / jnp.sqrt(jnp.maximum(v, 0.0) + 1e-5) * bg_ref[...] \
        + bb_ref[...]
    return jnp.maximum(hn, 0.0)


def _layer23_body(a0_ref, a1_ref, e0_ref, e1_ref, sm_ref, sq_ref,
                  bg_ref, bb_ref, epw_ref, epb_ref, lw_ref, lb_ref,
                  g0_ref, g1_ref):
    mask, dis, eam = _deg_dis_eam(e0_ref[...], e1_ref[...])
    h = _bn_relu(a0_ref, a1_ref, dis, sm_ref, sq_ref, bg_ref, bb_ref)
    t = h + jnp.dot(eam, epw_ref[...], preferred_element_type=jnp.float32) \
        + mask * epb_ref[...]
    h2 = jnp.dot(t, lw_ref[...], preferred_element_type=jnp.float32) + lb_ref[...]
    g = dis * h2
    g0_ref[...] = g[:, :DD]
    g1_ref[...] = g[:, DD:]


_layer23 = pl.pallas_call(
    _layer23_body,
    grid=(_NSTEPS,),
    in_specs=[
        pl.BlockSpec((_NBLK, DD), lambda i: (i, 0)),
        pl.BlockSpec((_NBLK, DD), lambda i: (i + _NSTEPS, 0)),
        pl.BlockSpec((_NBLK, 32), lambda i: (i, 0)),
        pl.BlockSpec((_NBLK, 32), lambda i: (i + _NSTEPS, 0)),
        pl.BlockSpec((1, HH), lambda i: (0, 0)),
        pl.BlockSpec((1, HH), lambda i: (0, 0)),
        pl.BlockSpec((1, HH), lambda i: (0, 0)),
        pl.BlockSpec((1, HH), lambda i: (0, 0)),
        pl.BlockSpec((ED, HH), lambda i: (0, 0)),
        pl.BlockSpec((1, HH), lambda i: (0, 0)),
        pl.BlockSpec((HH, HH), lambda i: (0, 0)),
        pl.BlockSpec((1, HH), lambda i: (0, 0)),
    ],
    out_specs=[
        pl.BlockSpec((_NBLK, DD), lambda i: (i, 0)),
        pl.BlockSpec((_NBLK, DD), lambda i: (i, 0)),
    ],
    out_shape=[
        jax.ShapeDtypeStruct((NN, DD), jnp.float32),
        jax.ShapeDtypeStruct((NN, DD), jnp.float32),
    ],
)


def _pool_body(a0_ref, a1_ref, e0_ref, e1_ref, sm_ref, sq_ref,
               bg_ref, bb_ref, b_ref, cn_ref, ow_ref, ob_ref,
               p_ref, out_ref):
    i = pl.program_id(0)

    @pl.when(i == 0)
    def _():
        p_ref[...] = jnp.zeros_like(p_ref)

    _, dis, _ = _deg_dis_eam(e0_ref[...], e1_ref[...])
    h = _bn_relu(a0_ref, a1_ref, dis, sm_ref, sq_ref, bg_ref, bb_ref)
    oh = _onehot(b_ref)
    p_ref[...] += _c00(oh, h)

    @pl.when(i == _NSTEPS - 1)
    def _():
        cnt = jnp.maximum(cn_ref[...], 1.0)
        pooled = p_ref[...] / cnt
        out_ref[...] = jnp.dot(pooled, ow_ref[...],
                               preferred_element_type=jnp.float32) + ob_ref[...]


_pool = pl.pallas_call(
    _pool_body,
    grid=(_NSTEPS,),
    in_specs=[
        pl.BlockSpec((_NBLK, DD), lambda i: (i, 0)),
        pl.BlockSpec((_NBLK, DD), lambda i: (i + _NSTEPS, 0)),
        pl.BlockSpec((_NBLK, 32), lambda i: (i, 0)),
        pl.BlockSpec((_NBLK, 32), lambda i: (i + _NSTEPS, 0)),
        pl.BlockSpec((1, HH), lambda i: (0, 0)),
        pl.BlockSpec((1, HH), lambda i: (0, 0)),
        pl.BlockSpec((1, HH), lambda i: (0, 0)),
        pl.BlockSpec((1, HH), lambda i: (0, 0)),
        pl.BlockSpec((_NBLK, 1), lambda i: (i, 0)),
        pl.BlockSpec((GG, 1), lambda i: (0, 0)),
        pl.BlockSpec((HH, OUTD), lambda i: (0, 0)),
        pl.BlockSpec((1, OUTD), lambda i: (0, 0)),
    ],
    out_specs=[
        pl.BlockSpec((GG, HH), lambda i: (0, 0)),
        pl.BlockSpec((GG, OUTD), lambda i: (0, 0)),
    ],
    out_shape=[
        jax.ShapeDtypeStruct((GG, HH), jnp.float32),
        jax.ShapeDtypeStruct((GG, OUTD), jnp.float32),
    ],
)


def kernel(x, edge_attr, lin_in_w, lin_in_b, edge_emb_w, edge_emb_b,
           ep1_w, ep1_b, lin1_w, lin1_b, bn1_g, bn1_b,
           ep2_w, ep2_b, lin2_w, lin2_b, bn2_g, bn2_b,
           ep3_w, ep3_b, lin3_w, lin3_b, bn3_g, bn3_b,
           out_w, out_b, edge_index, batch):
    row = edge_index[0]
    col = edge_index[1]
    b2d = batch.reshape(NN, 1)
    row2d = row.reshape(EE, 1)

    Sx, Qx, Cn, Cnr = _node_stats(x, b2d)
    S, Q, C = _edge_stats(edge_attr, row2d, Cnr)
    ea32 = _edge_embed(edge_attr, row2d, Cnr, S, Q, C,
                       edge_emb_w, edge_emb_b.reshape(1, ED))
    zeros32 = jnp.zeros((_LAST_STRIPE, 32), jnp.float32)
    eacc = _edge_scatter(ea32, col, zeros32)
    g0, g1 = _layer1(x, b2d, Sx, Qx, Cn, eacc, eacc,
                     lin_in_w, lin_in_b.reshape(1, HH),
                     ep1_w, ep1_b.reshape(1, HH),
                     lin1_w, lin1_b.reshape(1, HH))

    zerosD = jnp.zeros((_LAST_STRIPE, DD), jnp.float32)
    gst = jnp.concatenate([g0, g1], axis=0)
    oacc1 = _spmm(gst, row, col, zerosD)

    sm1, sq1 = _bnstats(oacc1, oacc1, eacc, eacc)
    g0, g1 = _layer23(oacc1, oacc1, eacc, eacc, sm1, sq1,
                      bn1_g.reshape(1, HH), bn1_b.reshape(1, HH),
                      ep2_w, ep2_b.reshape(1, HH),
                      lin2_w, lin2_b.reshape(1, HH))
    gst = jnp.concatenate([g0, g1], axis=0)
    oacc2 = _spmm(gst, row, col, zerosD)

    sm2, sq2 = _bnstats(oacc2, oacc2, eacc, eacc)
    g0, g1 = _layer23(oacc2, oacc2, eacc, eacc, sm2, sq2,
                      bn2_g.reshape(1, HH), bn2_b.reshape(1, HH),
                      ep3_w, ep3_b.reshape(1, HH),
                      lin3_w, lin3_b.reshape(1, HH))
    gst = jnp.concatenate([g0, g1], axis=0)
    oacc3 = _spmm(gst, row, col, zerosD)

    sm3, sq3 = _bnstats(oacc3, oacc3, eacc, eacc)
    _, out = _pool(oacc3, oacc3, eacc, eacc, sm3, sq3,
                   bn3_g.reshape(1, HH), bn3_b.reshape(1, HH),
                   b2d, Cn, out_w, out_b.reshape(1, OUTD))
    return out
